# baseline (speedup 1.0000x reference)
import functools

import jax
import jax.numpy as jnp
from jax.experimental import pallas as pl
from jax.experimental.pallas import tpu as pltpu

_BISECT_ITERS = 18


def _rsum(x):
    return jnp.sum(x, axis=1, keepdims=True)


def _match_kernel(bbox_ref, anc_ref, boxes_ref, labels_ref,
                  label_out, neg_out, bbox_out, *, n_batch, n_anchors,
                  n_targets):
    lane = jax.lax.broadcasted_iota(jnp.int32, (1, n_anchors), 1)

    anc = anc_ref[...]
    ax1 = anc[0]
    ay1 = anc[1]
    ax2 = anc[2]
    ay2 = anc[3]
    area_a = (ax2 - ax1) * (ay2 - ay1) + 1e-6

    best_iou = None
    best_lab = None
    mb = [None] * 4
    iou0 = None
    for t in range(n_targets):
        bx1 = boxes_ref[t, 0]
        by1 = boxes_ref[t, 1]
        bx2 = boxes_ref[t, 2]
        by2 = boxes_ref[t, 3]
        lab_t = labels_ref[t]
        ix1 = jnp.maximum(ax1, bx1)
        iy1 = jnp.maximum(ay1, by1)
        ix2 = jnp.minimum(ax2, bx2)
        iy2 = jnp.minimum(ay2, by2)
        inter = jnp.maximum(ix2 - ix1, 0.0) * jnp.maximum(iy2 - iy1, 0.0)
        area_b = (bx2 - bx1) * (by2 - by1)
        union = area_a + (area_b - inter)
        iou = inter / union
        if t == 0:
            iou0 = iou
            best_iou = iou
            best_lab = jnp.broadcast_to(lab_t, (n_batch, n_anchors))
            mb = [jnp.broadcast_to(b, (n_batch, n_anchors))
                  for b in (bx1, by1, bx2, by2)]
        else:
            upd = iou > best_iou
            best_iou = jnp.where(upd, iou, best_iou)
            best_lab = jnp.where(upd, lab_t, best_lab)
            mb = [jnp.where(upd, b, o)
                  for b, o in zip((bx1, by1, bx2, by2), mb)]

    pos = best_iou >= 0.5
    neg = best_iou < 0.4

    no_pos = jnp.logical_not(jnp.any(pos, axis=1, keepdims=True))
    m0 = jnp.max(iou0, axis=1, keepdims=True)
    cand = jnp.where(iou0 == m0, lane, n_anchors)
    bidx = jnp.min(cand, axis=1, keepdims=True)
    force = jnp.logical_and(no_pos, lane == bidx)
    pos = jnp.logical_or(pos, force)
    neg = jnp.logical_and(neg, jnp.logical_not(force))
    if n_targets > 1:
        best_lab = jnp.where(force, labels_ref[0], best_lab)
        mb = [jnp.where(force, boxes_ref[0, j], o)
              for j, o in enumerate(mb)]

    num_pos = _rsum(pos.astype(jnp.float32))

    label_out[...] = jnp.where(pos, best_lab, 0)
    neg_out[...] = neg.astype(jnp.float32)

    sl1 = jnp.zeros((n_batch, n_anchors), dtype=jnp.float32)
    for j in range(4):
        d = bbox_ref[j] - mb[j]
        ad = jnp.abs(d)
        sl1 = sl1 + jnp.where(ad < 1.0, 0.5 * ad * ad, ad - 0.5)
    bbox_out[...] = _rsum(jnp.where(pos, sl1, 0.0)) / num_pos


def _conf_kernel(conf_ref, label_ref, neg_ref, conf_out, *, n_batch,
                 n_anchors, n_classes):
    anchor_label = label_ref[...]
    pos = anchor_label > 0
    neg_f = neg_ref[...]
    neg = neg_f > 0.5
    num_pos = _rsum(pos.astype(jnp.float32))
    n_neg = _rsum(neg_f)
    k = jnp.minimum(3.0 * num_pos, n_neg)

    m = conf_ref[:, 0, :]
    for c in range(1, n_classes):
        m = jnp.maximum(m, conf_ref[:, c, :])
    picked = conf_ref[:, 0, :]
    s = jnp.exp(picked - m)
    for c in range(1, n_classes):
        logit = conf_ref[:, c, :]
        s = s + jnp.exp(logit - m)
        picked = jnp.where(anchor_label == c, logit, picked)
    ce = m + jnp.log(s) - picked

    ce_neg = jnp.where(neg, ce, -1.0)
    hi0 = jnp.maximum(jnp.max(ce_neg, axis=1, keepdims=True), 0.0)

    def bisect(_, carry):
        lo, hi = carry
        mid = 0.5 * (lo + hi)
        cnt = _rsum((ce_neg > mid).astype(jnp.float32))
        take = cnt >= k
        return jnp.where(take, mid, lo), jnp.where(take, hi, mid)

    lo, hi = jax.lax.fori_loop(
        0, _BISECT_ITERS, bisect, (jnp.zeros_like(hi0), hi0))
    thr = lo
    sel = ce_neg > thr
    cnt_gt = _rsum(sel.astype(jnp.float32))
    sum_gt = _rsum(jnp.where(sel, ce_neg, 0.0))
    topk = sum_gt + (k - cnt_gt) * thr

    pos_ce = _rsum(jnp.where(pos, ce, 0.0))
    conf_out[...] = (pos_ce + topk) / (num_pos + k)


def kernel(bbox_pred, conf_pred, anchors, target_boxes, target_labels,
           conf_weight=1.0, bbox_weight=1.0):
    B, N, C = conf_pred.shape
    T = target_boxes.shape[1]

    anc = anchors.T.reshape(4, 1, N)
    bbox_t = bbox_pred.transpose(2, 0, 1)
    conf_t = conf_pred.transpose(0, 2, 1)
    boxes_v = target_boxes.transpose(1, 2, 0).reshape(T, 4, B, 1)
    labels_v = target_labels.astype(jnp.int32).T.reshape(T, B, 1)

    match_body = functools.partial(_match_kernel, n_batch=B, n_anchors=N,
                                   n_targets=T)
    anchor_label, neg_f, bbox_out = pl.pallas_call(
        match_body,
        out_shape=[
            jax.ShapeDtypeStruct((B, N), jnp.int32),
            jax.ShapeDtypeStruct((B, N), jnp.float32),
            jax.ShapeDtypeStruct((B, 1), jnp.float32),
        ],
    )(bbox_t, anc, boxes_v, labels_v)

    conf_body = functools.partial(_conf_kernel, n_batch=B, n_anchors=N,
                                  n_classes=C)
    conf_out = pl.pallas_call(
        conf_body,
        out_shape=jax.ShapeDtypeStruct((B, 1), jnp.float32),
    )(conf_t, anchor_label, neg_f)

    conf_loss = jnp.sum(conf_out) / B
    bbox_loss = jnp.sum(bbox_out) / B
    total = conf_weight * conf_loss + bbox_weight * bbox_loss
    return total, conf_loss, bbox_loss


# speedup vs baseline: 1.5281x; 1.5281x over previous
import functools

import jax
import jax.numpy as jnp
from jax.experimental import pallas as pl
from jax.experimental.pallas import tpu as pltpu

_BISECT_ITERS = 16


def _rsum(x):
    return jnp.sum(x, axis=1, keepdims=True)


def _loss_kernel(conf_ref, bbox_ref, anc_ref, boxes_ref, labels_ref,
                 conf_out, bbox_out, *, n_batch, n_anchors, n_targets,
                 n_classes):
    lane = jax.lax.broadcasted_iota(jnp.int32, (1, n_anchors), 1)

    anc = anc_ref[...]
    ax1 = anc[0]
    ay1 = anc[1]
    ax2 = anc[2]
    ay2 = anc[3]
    area_a = (ax2 - ax1) * (ay2 - ay1) + 1e-6

    best_iou = None
    best_lab = None
    mb = [None] * 4
    iou0 = None
    for t in range(n_targets):
        bx1 = boxes_ref[t, 0]
        by1 = boxes_ref[t, 1]
        bx2 = boxes_ref[t, 2]
        by2 = boxes_ref[t, 3]
        lab_t = labels_ref[t]
        ix1 = jnp.maximum(ax1, bx1)
        iy1 = jnp.maximum(ay1, by1)
        ix2 = jnp.minimum(ax2, bx2)
        iy2 = jnp.minimum(ay2, by2)
        inter = jnp.maximum(ix2 - ix1, 0.0) * jnp.maximum(iy2 - iy1, 0.0)
        area_b = (bx2 - bx1) * (by2 - by1)
        union = area_a + (area_b - inter)
        iou = inter / union
        if t == 0:
            iou0 = iou
            best_iou = iou
            best_lab = jnp.broadcast_to(lab_t, (n_batch, n_anchors))
            mb = [jnp.broadcast_to(b, (n_batch, n_anchors))
                  for b in (bx1, by1, bx2, by2)]
        else:
            upd = iou > best_iou
            best_iou = jnp.where(upd, iou, best_iou)
            best_lab = jnp.where(upd, lab_t, best_lab)
            mb = [jnp.where(upd, b, o)
                  for b, o in zip((bx1, by1, bx2, by2), mb)]

    pos = best_iou >= 0.5
    neg = best_iou < 0.4

    no_pos = jnp.logical_not(jnp.any(pos, axis=1, keepdims=True))
    m0 = jnp.max(iou0, axis=1, keepdims=True)
    cand = jnp.where(iou0 == m0, lane, n_anchors)
    bidx = jnp.min(cand, axis=1, keepdims=True)
    force = jnp.logical_and(no_pos, lane == bidx)
    pos = jnp.logical_or(pos, force)
    neg = jnp.logical_and(neg, jnp.logical_not(force))
    if n_targets > 1:
        best_lab = jnp.where(force, labels_ref[0], best_lab)
        mb = [jnp.where(force, boxes_ref[0, j], o)
              for j, o in enumerate(mb)]

    num_pos = _rsum(pos.astype(jnp.float32))
    anchor_label = jnp.where(pos, best_lab, 0)

    m = conf_ref[0]
    for c in range(1, n_classes):
        m = jnp.maximum(m, conf_ref[c])
    picked = conf_ref[0]
    s = jnp.exp(picked - m)
    for c in range(1, n_classes):
        logit = conf_ref[c]
        s = s + jnp.exp(logit - m)
        picked = jnp.where(anchor_label == c, logit, picked)
    ce = m + jnp.log(s) - picked

    n_neg = _rsum(neg.astype(jnp.float32))
    k = jnp.minimum(3.0 * num_pos, n_neg)

    ce_neg = jnp.where(neg, ce, -1.0)
    hi0 = jnp.maximum(jnp.max(ce_neg, axis=1, keepdims=True), 0.0)

    def bisect(_, carry):
        lo, hi = carry
        mid = 0.5 * (lo + hi)
        cnt = _rsum((ce_neg > mid).astype(jnp.float32))
        take = cnt >= k
        return jnp.where(take, mid, lo), jnp.where(take, hi, mid)

    lo, hi = jax.lax.fori_loop(
        0, _BISECT_ITERS, bisect, (jnp.zeros_like(hi0), hi0))
    thr = lo
    sel = ce_neg > thr
    cnt_gt = _rsum(sel.astype(jnp.float32))
    sum_gt = _rsum(jnp.where(sel, ce_neg, 0.0))
    topk = sum_gt + (k - cnt_gt) * thr

    pos_ce = _rsum(jnp.where(pos, ce, 0.0))
    conf_loss = (pos_ce + topk) / (num_pos + k)

    sl1 = jnp.zeros((n_batch, n_anchors), dtype=jnp.float32)
    for j in range(4):
        d = bbox_ref[j] - mb[j]
        ad = jnp.abs(d)
        sl1 = sl1 + jnp.where(ad < 1.0, 0.5 * ad * ad, ad - 0.5)
    bbox_loss = _rsum(jnp.where(pos, sl1, 0.0)) / num_pos

    conf_out[...] = conf_loss
    bbox_out[...] = bbox_loss


def kernel(bbox_pred, conf_pred, anchors, target_boxes, target_labels,
           conf_weight=1.0, bbox_weight=1.0):
    B, N, C = conf_pred.shape
    T = target_boxes.shape[1]

    anc = anchors.T.reshape(4, 1, N)
    bbox_t = bbox_pred.transpose(2, 0, 1)
    conf_t = conf_pred.transpose(2, 0, 1)
    boxes_v = target_boxes.transpose(1, 2, 0).reshape(T, 4, B, 1)
    labels_v = target_labels.astype(jnp.int32).T.reshape(T, B, 1)

    body = functools.partial(_loss_kernel, n_batch=B, n_anchors=N,
                             n_targets=T, n_classes=C)
    conf_out, bbox_out = pl.pallas_call(
        body,
        out_shape=[
            jax.ShapeDtypeStruct((B, 1), jnp.float32),
            jax.ShapeDtypeStruct((B, 1), jnp.float32),
        ],
    )(conf_t, bbox_t, anc, boxes_v, labels_v)

    conf_loss = jnp.sum(conf_out) / B
    bbox_loss = jnp.sum(bbox_out) / B
    total = conf_weight * conf_loss + bbox_weight * bbox_loss
    return total, conf_loss, bbox_loss
